# baseline (speedup 1.0000x reference)
import functools

import jax
import jax.numpy as jnp
from jax import lax
from jax.experimental import pallas as pl
from jax.experimental.pallas import tpu as pltpu
from jax.experimental.pallas import tpu_sc as plsc

_SPARSITY = 0.5
_MOMENTUM = 0.1

_B1 = 12
_B2 = 8
_NB1 = 1 << _B1
_NB2 = 1 << _B2
_SH1 = 31 - _B1
_SH2 = _SH1 - _B2


def _sc_kth_bits(x2, *, n_rows, length, k, num_ch):
    info = plsc.get_sparse_core_info()
    nc, ns = info.num_cores, info.num_subcores
    nw = nc * ns
    ch_per_w = num_ch // nw
    nl = n_rows * length
    mesh = plsc.VectorSubcoreMesh(core_axis_name="c", subcore_axis_name="s")

    @functools.partial(
        pl.kernel,
        mesh=mesh,
        out_type=jax.ShapeDtypeStruct((num_ch,), jnp.int32),
        scratch_types=[
            pltpu.VMEM((nl,), jnp.float32),
            pltpu.VMEM((_NB1,), jnp.int32),
            pltpu.VMEM((_NB2,), jnp.int32),
            pltpu.VMEM((ch_per_w,), jnp.int32),
        ],
        compiler_params=pltpu.CompilerParams(needs_layout_passes=False),
    )
    def body(x_hbm, kth_hbm, buf, hist, hist2, kbuf):
        wid = lax.axis_index("s") * nc + lax.axis_index("c")
        ch0 = wid * ch_per_w
        ones = jnp.ones((16,), jnp.int32)
        zeros16 = jnp.zeros((16,), jnp.int32)
        lane = lax.iota(jnp.int32, 16)
        lane0 = lane == 0

        def chan_body(i, _):
            c = ch0 + i
            for n in range(n_rows):
                pltpu.sync_copy(
                    x_hbm.at[n * num_ch + c], buf.at[pl.ds(n * length, length)]
                )

            def zero1(j, _):
                hist[pl.ds(j * 16, 16)] = zeros16
                return 0

            lax.fori_loop(0, _NB1 // 16, zero1, 0)

            def pass1(j, _):
                v = buf[pl.ds(j * 16, 16)]
                b = plsc.bitcast(v, jnp.int32) & jnp.int32(0x7FFFFFFF)
                bk = lax.shift_right_logical(b, _SH1)
                plsc.addupdate_scatter(hist, [bk], ones)
                return 0

            lax.fori_loop(0, nl // 16, pass1, 0)

            def scan1(j, carry):
                bacc, rank, run = carry
                h = hist[pl.ds(j * 16, 16)]
                cs = plsc.cumsum(h)
                ind = (run + cs) < k
                bacc = bacc + jnp.sum(ind.astype(jnp.int32))
                rank = rank + jnp.sum(jnp.where(ind, h, 0))
                run = run + jnp.sum(h)
                return (bacc, rank, run)

            bsel, rank1, _ = lax.fori_loop(
                0, _NB1 // 16, scan1, (jnp.int32(0), jnp.int32(0), jnp.int32(0))
            )

            def zero2(j, _):
                hist2[pl.ds(j * 16, 16)] = zeros16
                return 0

            lax.fori_loop(0, _NB2 // 16, zero2, 0)

            def pass2(j, _):
                v = buf[pl.ds(j * 16, 16)]
                b = plsc.bitcast(v, jnp.int32) & jnp.int32(0x7FFFFFFF)
                bk = lax.shift_right_logical(b, _SH1)
                sub = lax.shift_right_logical(b, _SH2) & jnp.int32(_NB2 - 1)
                plsc.addupdate_scatter(hist2, [sub], ones, mask=bk == bsel)
                return 0

            lax.fori_loop(0, nl // 16, pass2, 0)

            k2 = k - rank1

            def scan2(j, carry):
                sacc, run = carry
                h = hist2[pl.ds(j * 16, 16)]
                cs = plsc.cumsum(h)
                ind = (run + cs) < k2
                sacc = sacc + jnp.sum(ind.astype(jnp.int32))
                run = run + jnp.sum(h)
                return (sacc, run)

            ssel, _ = lax.fori_loop(
                0, _NB2 // 16, scan2, (jnp.int32(0), jnp.int32(0))
            )

            bits = (
                lax.shift_left(bsel, _SH1)
                | lax.shift_left(ssel, _SH2)
                | jnp.int32(1 << (_SH2 - 1))
            )
            plsc.store_scatter(
                kbuf, [jnp.full((16,), i, jnp.int32)],
                jnp.full((16,), bits, jnp.int32), mask=lane0,
            )
            return 0

        lax.fori_loop(0, ch_per_w, chan_body, 0)
        pltpu.sync_copy(kbuf, kth_hbm.at[pl.ds(ch0, ch_per_w)])

    return body(x2)


def _mask_body(x_ref, t_ref, kb_ref, o_ref):
    kth = jax.lax.bitcast_convert_type(kb_ref[...], jnp.float32)
    thr = t_ref[...] * (1.0 - _MOMENTUM) + kth * _MOMENTUM
    xv = x_ref[...]
    o_ref[...] = jnp.where(jnp.abs(xv) > thr[None, :, None], xv, 0.0)


def kernel(x, thresholds):
    N, C, L = x.shape
    k = max(1, int(N * L * _SPARSITY))
    kth_bits = _sc_kth_bits(
        x.reshape(N * C, L), n_rows=N, length=L, k=k, num_ch=C
    )
    c_chunk = 128
    out = pl.pallas_call(
        _mask_body,
        grid=(C // c_chunk,),
        in_specs=[
            pl.BlockSpec((N, c_chunk, L), lambda i: (0, i, 0)),
            pl.BlockSpec((c_chunk,), lambda i: (i,)),
            pl.BlockSpec((c_chunk,), lambda i: (i,)),
        ],
        out_specs=pl.BlockSpec((N, c_chunk, L), lambda i: (0, i, 0)),
        out_shape=jax.ShapeDtypeStruct((N, C, L), jnp.float32),
        compiler_params=pltpu.CompilerParams(
            dimension_semantics=("arbitrary",),
        ),
    )(x, thresholds, kth_bits)
    return out


# speedup vs baseline: 1.2844x; 1.2844x over previous
import functools

import jax
import jax.numpy as jnp
from jax import lax
from jax.experimental import pallas as pl
from jax.experimental.pallas import tpu as pltpu
from jax.experimental.pallas import tpu_sc as plsc

_SPARSITY = 0.5
_MOMENTUM = 0.1

_NCOARSE = 256
_FBITS = 7
_NFINE = 1 << _FBITS
_SHF = 23 - _FBITS
_UNROLL = 8


def _sc_kth_bits(x2, *, n_rows, length, k, num_ch):
    info = plsc.get_sparse_core_info()
    nc, ns = info.num_cores, info.num_subcores
    nw = nc * ns
    ch_per_w = num_ch // nw
    nl = n_rows * length
    mesh = plsc.VectorSubcoreMesh(core_axis_name="c", subcore_axis_name="s")

    @functools.partial(
        pl.kernel,
        mesh=mesh,
        out_type=jax.ShapeDtypeStruct((num_ch,), jnp.int32),
        scratch_types=[
            pltpu.VMEM((2 * nl,), jnp.float32),
            pltpu.VMEM((_NCOARSE,), jnp.int32),
            pltpu.VMEM((_NFINE,), jnp.int32),
            pltpu.VMEM((ch_per_w,), jnp.int32),
            pltpu.SemaphoreType.DMA,
            pltpu.SemaphoreType.DMA,
        ],
        compiler_params=pltpu.CompilerParams(needs_layout_passes=False),
    )
    def body(x_hbm, kth_hbm, buf, hcoarse, hfine, kbuf, sem_a, sem_b):
        wid = lax.axis_index("s") * nc + lax.axis_index("c")
        ch0 = wid * ch_per_w
        ones = jnp.ones((16,), jnp.int32)
        zeros16 = jnp.zeros((16,), jnp.int32)
        lane0 = lax.iota(jnp.int32, 16) == 0
        mask31 = jnp.int32(0x7FFFFFFF)

        def dma_descs(c, slot, sem):
            return [
                pltpu.make_async_copy(
                    x_hbm.at[n * num_ch + c],
                    buf.at[pl.ds(slot * nl + n * length, length)],
                    sem,
                )
                for n in range(n_rows)
            ]

        def start_dma(c, slot, sem):
            for d in dma_descs(c, slot, sem):
                d.start()

        def wait_dma(c, slot, sem):
            for d in dma_descs(c, slot, sem):
                d.wait()

        def scan_hist(href, nvec, kk):
            sel = jnp.int32(0)
            rank = jnp.int32(0)
            run = jnp.int32(0)
            for j in range(nvec):
                h = href[pl.ds(j * 16, 16)]
                cs = plsc.cumsum(h)
                ind = (run + cs) < kk
                sel = sel + jnp.sum(ind.astype(jnp.int32))
                rank = rank + jnp.sum(jnp.where(ind, h, 0))
                run = run + jnp.sum(h)
            return sel, rank

        def process(slot, i_local):
            base = slot * nl
            for j in range(_NCOARSE // 16):
                hcoarse[pl.ds(j * 16, 16)] = zeros16

            def p1(j, _):
                v = buf[pl.ds(base + j * 16, 16)]
                b = plsc.bitcast(v, jnp.int32) & mask31
                plsc.addupdate_scatter(
                    hcoarse, [lax.shift_right_logical(b, 23)], ones
                )
                return 0

            lax.fori_loop(0, nl // 16, p1, 0, unroll=_UNROLL)

            esel, rank1 = scan_hist(hcoarse, _NCOARSE // 16, k)

            for j in range(_NFINE // 16):
                hfine[pl.ds(j * 16, 16)] = zeros16

            def p2(j, _):
                v = buf[pl.ds(base + j * 16, 16)]
                b = plsc.bitcast(v, jnp.int32) & mask31
                ce = lax.shift_right_logical(b, 23)
                fe = lax.shift_right_logical(b, _SHF) & jnp.int32(_NFINE - 1)
                plsc.addupdate_scatter(hfine, [fe], ones, mask=ce == esel)
                return 0

            lax.fori_loop(0, nl // 16, p2, 0, unroll=_UNROLL)

            fsel, _ = scan_hist(hfine, _NFINE // 16, k - rank1)

            bits = (
                lax.shift_left(esel, 23)
                | lax.shift_left(fsel, _SHF)
                | jnp.int32(1 << (_SHF - 1))
            )
            plsc.store_scatter(
                kbuf, [jnp.full((16,), i_local, jnp.int32)],
                jnp.full((16,), bits, jnp.int32), mask=lane0,
            )

        start_dma(ch0, 0, sem_a)
        start_dma(ch0 + 1, 1, sem_b)

        def pair_body(i, _):
            ca = ch0 + 2 * i
            wait_dma(ca, 0, sem_a)
            process(0, 2 * i)

            @pl.when(i < ch_per_w // 2 - 1)
            def _prefetch_a():
                start_dma(ca + 2, 0, sem_a)

            wait_dma(ca + 1, 1, sem_b)
            process(1, 2 * i + 1)

            @pl.when(i < ch_per_w // 2 - 1)
            def _prefetch_b():
                start_dma(ca + 3, 1, sem_b)

            return 0

        lax.fori_loop(0, ch_per_w // 2, pair_body, 0)
        pltpu.sync_copy(kbuf, kth_hbm.at[pl.ds(ch0, ch_per_w)])

    return body(x2)


def _mask_body(x_ref, t_ref, kb_ref, o_ref):
    kth = jax.lax.bitcast_convert_type(kb_ref[...], jnp.float32)
    thr = t_ref[...] * (1.0 - _MOMENTUM) + kth * _MOMENTUM
    xv = x_ref[...]
    o_ref[...] = jnp.where(jnp.abs(xv) > thr[None, :, None], xv, 0.0)


def kernel(x, thresholds):
    N, C, L = x.shape
    k = max(1, int(N * L * _SPARSITY))
    kth_bits = _sc_kth_bits(
        x.reshape(N * C, L), n_rows=N, length=L, k=k, num_ch=C
    )
    c_chunk = 128
    out = pl.pallas_call(
        _mask_body,
        grid=(C // c_chunk,),
        in_specs=[
            pl.BlockSpec((N, c_chunk, L), lambda i: (0, i, 0)),
            pl.BlockSpec((c_chunk,), lambda i: (i,)),
            pl.BlockSpec((c_chunk,), lambda i: (i,)),
        ],
        out_specs=pl.BlockSpec((N, c_chunk, L), lambda i: (0, i, 0)),
        out_shape=jax.ShapeDtypeStruct((N, C, L), jnp.float32),
        compiler_params=pltpu.CompilerParams(
            dimension_semantics=("arbitrary",),
        ),
    )(x, thresholds, kth_bits)
    return out


# speedup vs baseline: 1.8516x; 1.4416x over previous
import functools

import jax
import jax.numpy as jnp
from jax import lax
from jax.experimental import pallas as pl
from jax.experimental.pallas import tpu as pltpu
from jax.experimental.pallas import tpu_sc as plsc

_SPARSITY = 0.5
_MOMENTUM = 0.1

_TBITS = 13
_MBITS = _TBITS - 8
_NBUCK = 1 << _TBITS
_SHF = 23 - _MBITS
_NEXP = 256
_NFINE = 1 << _MBITS
_UNROLL = 8


def _sc_kth_bits(x2, *, n_rows, length, k, num_ch):
    info = plsc.get_sparse_core_info()
    nc, ns = info.num_cores, info.num_subcores
    nw = nc * ns
    ch_per_w = num_ch // nw
    nl = n_rows * length
    mesh = plsc.VectorSubcoreMesh(core_axis_name="c", subcore_axis_name="s")

    @functools.partial(
        pl.kernel,
        mesh=mesh,
        out_type=jax.ShapeDtypeStruct((num_ch,), jnp.int32),
        scratch_types=[
            pltpu.VMEM((2 * nl,), jnp.float32),
            pltpu.VMEM((_NBUCK,), jnp.int32),
            pltpu.VMEM((ch_per_w,), jnp.int32),
            pltpu.SemaphoreType.DMA,
            pltpu.SemaphoreType.DMA,
        ],
        compiler_params=pltpu.CompilerParams(needs_layout_passes=False),
    )
    def body(x_hbm, kth_hbm, buf, hist, kbuf, sem_a, sem_b):
        wid = lax.axis_index("s") * nc + lax.axis_index("c")
        ch0 = wid * ch_per_w
        ones = jnp.ones((16,), jnp.int32)
        zeros16 = jnp.zeros((16,), jnp.int32)
        iota16 = lax.iota(jnp.int32, 16)
        lane0 = iota16 == 0
        stride32 = iota16 * _NFINE

        def z0(j, _):
            hist[pl.ds(j * 16, 16)] = zeros16
            return 0

        lax.fori_loop(0, _NBUCK // 16, z0, 0, unroll=_UNROLL)

        def dma_descs(c, slot, sem):
            return [
                pltpu.make_async_copy(
                    x_hbm.at[n * num_ch + c],
                    buf.at[pl.ds(slot * nl + n * length, length)],
                    sem,
                )
                for n in range(n_rows)
            ]

        def start_dma(c, slot, sem):
            for d in dma_descs(c, slot, sem):
                d.start()

        def wait_dma(c, slot, sem):
            for d in dma_descs(c, slot, sem):
                d.wait()

        def process(slot, i_local):
            base = slot * nl

            def p1(j, _):
                v = buf[pl.ds(base + j * 16, 16)]
                b = lax.shift_right_logical(
                    plsc.bitcast(v, jnp.int32), _SHF
                ) & jnp.int32(_NBUCK - 1)
                plsc.addupdate_scatter(hist, [b], ones)
                return 0

            lax.fori_loop(0, nl // 16, p1, 0, unroll=_UNROLL)

            esel = jnp.int32(0)
            rank1 = jnp.int32(0)
            run = jnp.int32(0)
            for g in range(_NEXP // 16):
                acc = zeros16
                for f in range(_NFINE):
                    acc = acc + plsc.load_gather(
                        hist, [stride32 + jnp.int32(g * 16 * _NFINE + f)]
                    )
                cs = plsc.cumsum(acc)
                ind = (run + cs) < k
                esel = esel + jnp.sum(ind.astype(jnp.int32))
                rank1 = rank1 + jnp.sum(jnp.where(ind, acc, 0))
                run = run + jnp.sum(acc)

            kk = k - rank1
            fbase = esel * _NFINE
            f0 = plsc.load_gather(hist, [fbase + iota16])
            f1 = plsc.load_gather(hist, [fbase + jnp.int32(16) + iota16])
            cs0 = plsc.cumsum(f0)
            ind0 = cs0 < kk
            fsel = jnp.sum(ind0.astype(jnp.int32))
            run0 = jnp.sum(f0)
            cs1 = plsc.cumsum(f1)
            ind1 = (run0 + cs1) < kk
            fsel = fsel + jnp.sum(ind1.astype(jnp.int32))

            bits = (
                lax.shift_left(esel, 23)
                | lax.shift_left(fsel, _SHF)
                | jnp.int32(1 << (_SHF - 1))
            )
            plsc.store_scatter(
                kbuf, [jnp.full((16,), i_local, jnp.int32)],
                jnp.full((16,), bits, jnp.int32), mask=lane0,
            )

            def zz(j, _):
                hist[pl.ds(j * 16, 16)] = zeros16
                return 0

            lax.fori_loop(0, _NBUCK // 16, zz, 0, unroll=_UNROLL)

        start_dma(ch0, 0, sem_a)
        start_dma(ch0 + 1, 1, sem_b)

        def pair_body(i, _):
            ca = ch0 + 2 * i
            wait_dma(ca, 0, sem_a)
            process(0, 2 * i)

            @pl.when(i < ch_per_w // 2 - 1)
            def _prefetch_a():
                start_dma(ca + 2, 0, sem_a)

            wait_dma(ca + 1, 1, sem_b)
            process(1, 2 * i + 1)

            @pl.when(i < ch_per_w // 2 - 1)
            def _prefetch_b():
                start_dma(ca + 3, 1, sem_b)

            return 0

        lax.fori_loop(0, ch_per_w // 2, pair_body, 0)
        pltpu.sync_copy(kbuf, kth_hbm.at[pl.ds(ch0, ch_per_w)])

    return body(x2)


def _mask_body(x_ref, t_ref, kb_ref, o_ref):
    kth = jax.lax.bitcast_convert_type(kb_ref[...], jnp.float32)
    thr = t_ref[...] * (1.0 - _MOMENTUM) + kth * _MOMENTUM
    xv = x_ref[...]
    o_ref[...] = jnp.where(jnp.abs(xv) > thr[None, :, None], xv, 0.0)


def kernel(x, thresholds):
    N, C, L = x.shape
    k = max(1, int(N * L * _SPARSITY))
    kth_bits = _sc_kth_bits(
        x.reshape(N * C, L), n_rows=N, length=L, k=k, num_ch=C
    )
    c_chunk = 128
    out = pl.pallas_call(
        _mask_body,
        grid=(C // c_chunk,),
        in_specs=[
            pl.BlockSpec((N, c_chunk, L), lambda i: (0, i, 0)),
            pl.BlockSpec((c_chunk,), lambda i: (i,)),
            pl.BlockSpec((c_chunk,), lambda i: (i,)),
        ],
        out_specs=pl.BlockSpec((N, c_chunk, L), lambda i: (0, i, 0)),
        out_shape=jax.ShapeDtypeStruct((N, C, L), jnp.float32),
        compiler_params=pltpu.CompilerParams(
            dimension_semantics=("arbitrary",),
        ),
    )(x, thresholds, kth_bits)
    return out
